# baseline (speedup 1.0000x reference)
import functools

import jax
import jax.numpy as jnp
from jax import lax
from jax.experimental import pallas as pl
from jax.experimental.pallas import tpu as pltpu
from jax.experimental.pallas import tpu_sc as plsc

N = 10000
E = 320000
D = 128

NC = 2
NS = 16
NW = NC * NS
CHUNK = 128
NCHUNK = E // CHUNK
GRP = 10
NGRP = NCHUNK // GRP
SCH = 128
SGRP = 5
SNGRP = E // SCH // SGRP
RS = 624
TAIL = N - NS * RS
DEGW = 128


def _sc_mesh():
  return plsc.VectorSubcoreMesh(core_axis_name="c", subcore_axis_name="s")


def _copy_idx_row(src_ref, ci, dst_ref, width):
  for k in range(width // 16):
    dst_ref[pl.ds(k * 16, 16)] = src_ref[ci, pl.ds(k * 16, 16)]


def _zero_and_writeback(c, s, acc_sh, zeros_hbm, out_hbm, scatter_loop):
  pltpu.sync_copy(zeros_hbm, acc_sh.at[pl.ds(s * RS, RS)])

  @pl.when(s == NS - 1)
  def _():
    pltpu.sync_copy(zeros_hbm.at[pl.ds(0, TAIL)],
                    acc_sh.at[pl.ds(NS * RS, TAIL)])

  plsc.subcore_barrier()
  scatter_loop()
  plsc.subcore_barrier()
  pltpu.sync_copy(acc_sh.at[pl.ds(s * RS, RS)],
                  out_hbm.at[c, pl.ds(s * RS, RS)])

  @pl.when(s == NS - 1)
  def _():
    pltpu.sync_copy(acc_sh.at[pl.ds(NS * RS, TAIL)],
                    out_hbm.at[c, pl.ds(NS * RS, TAIL)])


@functools.partial(
    pl.kernel,
    out_type=jax.ShapeDtypeStruct((NC, N, DEGW), jnp.float32),
    mesh=_sc_mesh(),
    scratch_types=[
        pltpu.VMEM((GRP, CHUNK), jnp.int32),
        pltpu.VMEM((CHUNK,), jnp.int32),
        pltpu.VMEM((CHUNK,), jnp.int32),
        pltpu.VMEM((CHUNK, DEGW), jnp.float32),
        pltpu.VMEM_SHARED((N, DEGW), jnp.float32),
        pltpu.SemaphoreType.DMA,
        pltpu.SemaphoreType.DMA,
    ],
)
def _sc_degree(dst_hbm, ones_hbm, zeros_hbm, out_hbm, didx, dbuf0, dbuf1,
               ones_v, acc_sh, ssem0, ssem1):
  c = lax.axis_index("c")
  s = lax.axis_index("s")
  w = s * NC + c
  pltpu.sync_copy(ones_hbm, ones_v)
  dbufs = (dbuf0, dbuf1)
  ssems = (ssem0, ssem1)

  def loop():
    @pl.loop(w, NGRP, step=NW)
    def _(g):
      pltpu.sync_copy(dst_hbm.at[g], didx)
      pending = [None, None]
      for ci in range(GRP):
        b = ci % 2
        if pending[b] is not None:
          pending[b].wait()
        _copy_idx_row(didx, ci, dbufs[b], CHUNK)
        pending[b] = pltpu.async_copy(ones_v, acc_sh.at[dbufs[b]], ssems[b],
                                      add=True)
      for p in pending:
        if p is not None:
          p.wait()

  _zero_and_writeback(c, s, acc_sh, zeros_hbm, out_hbm, loop)


@functools.partial(
    pl.kernel,
    out_type=jax.ShapeDtypeStruct((NC, N, D), jnp.float32),
    mesh=_sc_mesh(),
    scratch_types=[
        pltpu.VMEM((SGRP * SCH,), jnp.int32),
        pltpu.VMEM((SGRP, SCH), jnp.int32),
        [pltpu.VMEM((SCH,), jnp.int32)] * 2,
        pltpu.VMEM((3, SCH, D), jnp.float32),
        pltpu.VMEM_SHARED((N, D), jnp.float32),
        [pltpu.SemaphoreType.DMA] * 3,
        [pltpu.SemaphoreType.DMA] * 2,
    ],
)
def _sc_scatter(y_hbm, src_hbm, dst_hbm, zeros_hbm, out_hbm,
                sidx, didx, dbufs, rows, acc_sh, gsems, ssems):
  c = lax.axis_index("c")
  s = lax.axis_index("s")
  w = s * NC + c

  def loop():
    @pl.loop(w, SNGRP, step=NW)
    def _(g):
      pltpu.sync_copy(src_hbm.at[pl.ds(g * SGRP * SCH, SGRP * SCH)], sidx)
      pltpu.sync_copy(dst_hbm.at[g], didx)

      def gather(ci):
        b = ci % 3
        return pltpu.async_copy(
            y_hbm.at[sidx.at[pl.ds(ci * SCH, SCH)]],
            rows.at[b], gsems[b])

      pend_g = {0: gather(0)}
      pend_s = [None, None, None]
      for ci in range(SGRP):
        b = ci % 3
        nb = (ci + 1) % 3
        if pend_s[nb] is not None:
          pend_s[nb].wait()
          pend_s[nb] = None
        if ci + 1 < SGRP:
          pend_g[ci + 1] = gather(ci + 1)
        _copy_idx_row(didx, ci, dbufs[ci % 2], SCH)
        pend_g.pop(ci).wait()
        pend_s[b] = pltpu.async_copy(rows.at[b], acc_sh.at[dbufs[ci % 2]],
                                     ssems[ci % 2], add=True)
      for p in pend_s:
        if p is not None:
          p.wait()

  _zero_and_writeback(c, s, acc_sh, zeros_hbm, out_hbm, loop)


BM = 2000


def _dinv_block(degp_ref):
  deg = degp_ref[0, :, 0:1] + degp_ref[1, :, 0:1] + 2.0
  return lax.rsqrt(deg)


def _tc_matmul(x, w):
  def body(x_ref, w_ref, o_ref):
    o_ref[...] = jnp.dot(x_ref[...], w_ref[...],
                         preferred_element_type=jnp.float32)

  return pl.pallas_call(
      body,
      grid=(N // BM,),
      in_specs=[
          pl.BlockSpec((BM, D), lambda i: (i, 0)),
          pl.BlockSpec((D, D), lambda i: (0, 0)),
      ],
      out_specs=pl.BlockSpec((BM, D), lambda i: (i, 0)),
      out_shape=jax.ShapeDtypeStruct((N, D), jnp.float32),
  )(x, w)


def _tc_scale(xw, degp):
  def body(xw_ref, degp_ref, y_ref):
    y_ref[...] = xw_ref[...] * _dinv_block(degp_ref)

  return pl.pallas_call(
      body,
      grid=(N // BM,),
      in_specs=[
          pl.BlockSpec((BM, D), lambda i: (i, 0)),
          pl.BlockSpec((NC, BM, DEGW), lambda i: (0, i, 0)),
      ],
      out_specs=pl.BlockSpec((BM, D), lambda i: (i, 0)),
      out_shape=jax.ShapeDtypeStruct((N, D), jnp.float32),
  )(xw, degp)


def _tc_combine_mm(part, xw, degp, b, w_next):
  def body(part_ref, xw_ref, degp_ref, b_ref, w_ref, xwn_ref, yn_ref):
    dinv = _dinv_block(degp_ref)
    agg = (part_ref[0] + part_ref[1]) * dinv
    xl = jax.nn.relu(agg + xw_ref[...] * (2.0 * dinv * dinv) + b_ref[...])
    xwn = jnp.dot(xl, w_ref[...], preferred_element_type=jnp.float32)
    xwn_ref[...] = xwn
    yn_ref[...] = xwn * dinv

  return pl.pallas_call(
      body,
      grid=(N // BM,),
      in_specs=[
          pl.BlockSpec((NC, BM, D), lambda i: (0, i, 0)),
          pl.BlockSpec((BM, D), lambda i: (i, 0)),
          pl.BlockSpec((NC, BM, DEGW), lambda i: (0, i, 0)),
          pl.BlockSpec((1, D), lambda i: (0, 0)),
          pl.BlockSpec((D, D), lambda i: (0, 0)),
      ],
      out_specs=[
          pl.BlockSpec((BM, D), lambda i: (i, 0)),
          pl.BlockSpec((BM, D), lambda i: (i, 0)),
      ],
      out_shape=[
          jax.ShapeDtypeStruct((N, D), jnp.float32),
          jax.ShapeDtypeStruct((N, D), jnp.float32),
      ],
  )(part, xw, degp, b, w_next)


def _tc_head(part, xw, degp, b, wstack, bstack, wco, wlin, blin):
  def body(part_ref, xw_ref, degp_ref, b_ref, ws_ref, bs_ref, wco_ref,
           wlin_ref, blin_ref, o_ref):
    dinv = _dinv_block(degp_ref)
    agg = (part_ref[0] + part_ref[1]) * dinv
    x2 = jax.nn.relu(agg + xw_ref[...] * (2.0 * dinv * dinv) + b_ref[...])

    def mm(v, k):
      return jnp.dot(v, ws_ref[k], preferred_element_type=jnp.float32)

    i1 = jax.nn.sigmoid(mm(x2, 0) + bs_ref[0])
    t1 = jnp.tanh(mm(x2, 1) + bs_ref[1])
    c1 = i1 * t1
    o1 = jax.nn.sigmoid(mm(x2, 2) + bs_ref[2] + wco_ref[0] * c1)
    h1 = jax.nn.relu(o1 * jnp.tanh(c1))
    i2 = jax.nn.sigmoid(mm(h1, 3) + bs_ref[3])
    t2 = jnp.tanh(mm(h1, 4) + bs_ref[4])
    c2 = i2 * t2
    o2 = jax.nn.sigmoid(mm(h1, 5) + bs_ref[5] + wco_ref[1] * c2)
    h2 = jax.nn.relu(o2 * jnp.tanh(c2))
    i3 = jax.nn.sigmoid(mm(h2, 6) + bs_ref[6])
    t3 = jnp.tanh(mm(h2, 7) + bs_ref[7])
    c3 = jax.nn.relu(i3 * t3)
    o_ref[...] = (jnp.dot(c3, wlin_ref[...],
                          preferred_element_type=jnp.float32)
                  + blin_ref[...])

  return pl.pallas_call(
      body,
      grid=(N // BM,),
      in_specs=[
          pl.BlockSpec((NC, BM, D), lambda i: (0, i, 0)),
          pl.BlockSpec((BM, D), lambda i: (i, 0)),
          pl.BlockSpec((NC, BM, DEGW), lambda i: (0, i, 0)),
          pl.BlockSpec((1, D), lambda i: (0, 0)),
          pl.BlockSpec((8, D, D), lambda i: (0, 0, 0)),
          pl.BlockSpec((8, 1, D), lambda i: (0, 0, 0)),
          pl.BlockSpec((2, 1, D), lambda i: (0, 0, 0)),
          pl.BlockSpec((D, 3), lambda i: (0, 0)),
          pl.BlockSpec((1, 3), lambda i: (0, 0)),
      ],
      out_specs=pl.BlockSpec((BM, 3), lambda i: (i, 0)),
      out_shape=jax.ShapeDtypeStruct((N, 3), jnp.float32),
  )(part, xw, degp, b, wstack, bstack, wco, wlin, blin)


def kernel(x, edge_index, params):
  src = edge_index[0]
  dst = edge_index[1]
  dst3 = dst.reshape(NGRP, GRP, CHUNK)
  dst3s = dst.reshape(SNGRP, SGRP, SCH)

  ones_deg = jnp.ones((CHUNK, DEGW), jnp.float32)
  zeros_deg = jnp.zeros((RS, DEGW), jnp.float32)
  zeros_rows = jnp.zeros((RS, D), jnp.float32)

  xw1 = _tc_matmul(x, params['W1'])
  degp = _sc_degree(dst3, ones_deg, zeros_deg)
  y1 = _tc_scale(xw1, degp)
  p1 = _sc_scatter(y1, src, dst3s, zeros_rows)

  b1 = params['b1'].reshape(1, D)
  b2 = params['b2'].reshape(1, D)
  xw2, y2 = _tc_combine_mm(p1, xw1, degp, b1, params['W2'])
  p2 = _sc_scatter(y2, src, dst3s, zeros_rows)

  l1, l2, l3 = params['lstm1'], params['lstm2'], params['lstm3']
  wstack = jnp.stack([
      l1['Wxi'], l1['Wxc'], l1['Wxo'],
      l2['Wxi'], l2['Wxc'], l2['Wxo'],
      l3['Wxi'], l3['Wxc'],
  ])
  bstack = jnp.stack([
      l1['bxi'] + l1['bhi'] + l1['bi'],
      l1['bxc'] + l1['bhc'] + l1['bc'],
      l1['bxo'] + l1['bho'] + l1['bo'],
      l2['bxi'] + l2['bhi'] + l2['bi'],
      l2['bxc'] + l2['bhc'] + l2['bc'],
      l2['bxo'] + l2['bho'] + l2['bo'],
      l3['bxi'] + l3['bhi'] + l3['bi'],
      l3['bxc'] + l3['bhc'] + l3['bc'],
  ]).reshape(8, 1, D)
  wco = jnp.stack([l1['wco'], l2['wco']]).reshape(2, 1, D)
  blin = params['blin'].reshape(1, 3)

  return _tc_head(p2, xw2, degp, b2, wstack, bstack, wco,
                  params['Wlin'], blin)


# speedup vs baseline: 1.0720x; 1.0720x over previous
import functools

import jax
import jax.numpy as jnp
from jax import lax
from jax.experimental import pallas as pl
from jax.experimental.pallas import tpu as pltpu
from jax.experimental.pallas import tpu_sc as plsc

N = 10000
E = 320000
D = 128

NC = 2
NS = 16
NW = NC * NS
CHUNK = 128
NCHUNK = E // CHUNK
GRP = 10
NGRP = NCHUNK // GRP
SCH = 128
SGRP = 10
SNGRP = E // SCH // SGRP
RS = 624
TAIL = N - NS * RS
DEGW = 128


def _sc_mesh():
  return plsc.VectorSubcoreMesh(core_axis_name="c", subcore_axis_name="s")


def _copy_idx_row(src_ref, ci, dst_ref, width):
  for k in range(width // 16):
    dst_ref[pl.ds(k * 16, 16)] = src_ref[ci, pl.ds(k * 16, 16)]


def _zero_and_writeback(c, s, acc_sh, zeros_hbm, out_hbm, scatter_loop):
  pltpu.sync_copy(zeros_hbm, acc_sh.at[pl.ds(s * RS, RS)])

  @pl.when(s == NS - 1)
  def _():
    pltpu.sync_copy(zeros_hbm.at[pl.ds(0, TAIL)],
                    acc_sh.at[pl.ds(NS * RS, TAIL)])

  plsc.subcore_barrier()
  scatter_loop()
  plsc.subcore_barrier()
  pltpu.sync_copy(acc_sh.at[pl.ds(s * RS, RS)],
                  out_hbm.at[c, pl.ds(s * RS, RS)])

  @pl.when(s == NS - 1)
  def _():
    pltpu.sync_copy(acc_sh.at[pl.ds(NS * RS, TAIL)],
                    out_hbm.at[c, pl.ds(NS * RS, TAIL)])


@functools.partial(
    pl.kernel,
    out_type=jax.ShapeDtypeStruct((NC, N, DEGW), jnp.float32),
    mesh=_sc_mesh(),
    scratch_types=[
        pltpu.VMEM((GRP, CHUNK), jnp.int32),
        pltpu.VMEM((CHUNK,), jnp.int32),
        pltpu.VMEM((CHUNK,), jnp.int32),
        pltpu.VMEM((CHUNK, DEGW), jnp.float32),
        pltpu.VMEM_SHARED((N, DEGW), jnp.float32),
        pltpu.SemaphoreType.DMA,
        pltpu.SemaphoreType.DMA,
    ],
)
def _sc_degree(dst_hbm, ones_hbm, zeros_hbm, out_hbm, didx, dbuf0, dbuf1,
               ones_v, acc_sh, ssem0, ssem1):
  c = lax.axis_index("c")
  s = lax.axis_index("s")
  w = s * NC + c
  pltpu.sync_copy(ones_hbm, ones_v)
  dbufs = (dbuf0, dbuf1)
  ssems = (ssem0, ssem1)

  def loop():
    @pl.loop(w, NGRP, step=NW)
    def _(g):
      pltpu.sync_copy(dst_hbm.at[g], didx)
      pending = [None, None]
      for ci in range(GRP):
        b = ci % 2
        if pending[b] is not None:
          pending[b].wait()
        _copy_idx_row(didx, ci, dbufs[b], CHUNK)
        pending[b] = pltpu.async_copy(ones_v, acc_sh.at[dbufs[b]], ssems[b],
                                      add=True)
      for p in pending:
        if p is not None:
          p.wait()

  _zero_and_writeback(c, s, acc_sh, zeros_hbm, out_hbm, loop)


@functools.partial(
    pl.kernel,
    out_type=jax.ShapeDtypeStruct((NC, N, D), jnp.float32),
    mesh=_sc_mesh(),
    scratch_types=[
        pltpu.VMEM((SGRP * SCH,), jnp.int32),
        pltpu.VMEM((SGRP, SCH), jnp.int32),
        pltpu.VMEM((SCH,), jnp.int32),
        pltpu.VMEM((2, SCH, D), jnp.float32),
        pltpu.VMEM_SHARED((N, D), jnp.float32),
        [pltpu.SemaphoreType.DMA] * 2,
    ],
)
def _sc_scatter(y_hbm, src_hbm, dst_hbm, zeros_hbm, out_hbm,
                sidx, didx, dbuf, rows, acc_sh, gsems):
  c = lax.axis_index("c")
  s = lax.axis_index("s")
  w = s * NC + c

  def loop():
    @pl.loop(w, SNGRP, step=NW)
    def _(g):
      pltpu.sync_copy(src_hbm.at[pl.ds(g * SGRP * SCH, SGRP * SCH)], sidx)
      pltpu.sync_copy(dst_hbm.at[g], didx)

      def gather(ci):
        b = ci % 2
        return pltpu.async_copy(
            y_hbm.at[sidx.at[pl.ds(ci * SCH, SCH)]],
            rows.at[b], gsems[b])

      pending = gather(0)
      for ci in range(SGRP):
        nxt = gather(ci + 1) if ci + 1 < SGRP else None
        _copy_idx_row(didx, ci, dbuf, SCH)
        pending.wait()
        pltpu.sync_copy(rows.at[ci % 2], acc_sh.at[dbuf], add=True)
        pending = nxt

  _zero_and_writeback(c, s, acc_sh, zeros_hbm, out_hbm, loop)


BM = 2000


def _dinv_block(degp_ref):
  deg = degp_ref[0, :, 0:1] + degp_ref[1, :, 0:1] + 2.0
  return lax.rsqrt(deg)


def _tc_matmul(x, w):
  def body(x_ref, w_ref, o_ref):
    o_ref[...] = jnp.dot(x_ref[...], w_ref[...],
                         preferred_element_type=jnp.float32)

  return pl.pallas_call(
      body,
      grid=(N // BM,),
      in_specs=[
          pl.BlockSpec((BM, D), lambda i: (i, 0)),
          pl.BlockSpec((D, D), lambda i: (0, 0)),
      ],
      out_specs=pl.BlockSpec((BM, D), lambda i: (i, 0)),
      out_shape=jax.ShapeDtypeStruct((N, D), jnp.float32),
  )(x, w)


def _tc_scale(xw, degp):
  def body(xw_ref, degp_ref, y_ref):
    y_ref[...] = xw_ref[...] * _dinv_block(degp_ref)

  return pl.pallas_call(
      body,
      grid=(N // BM,),
      in_specs=[
          pl.BlockSpec((BM, D), lambda i: (i, 0)),
          pl.BlockSpec((NC, BM, DEGW), lambda i: (0, i, 0)),
      ],
      out_specs=pl.BlockSpec((BM, D), lambda i: (i, 0)),
      out_shape=jax.ShapeDtypeStruct((N, D), jnp.float32),
  )(xw, degp)


def _tc_combine_mm(part, xw, degp, b, w_next):
  def body(part_ref, xw_ref, degp_ref, b_ref, w_ref, xwn_ref, yn_ref):
    dinv = _dinv_block(degp_ref)
    agg = (part_ref[0] + part_ref[1]) * dinv
    xl = jax.nn.relu(agg + xw_ref[...] * (2.0 * dinv * dinv) + b_ref[...])
    xwn = jnp.dot(xl, w_ref[...], preferred_element_type=jnp.float32)
    xwn_ref[...] = xwn
    yn_ref[...] = xwn * dinv

  return pl.pallas_call(
      body,
      grid=(N // BM,),
      in_specs=[
          pl.BlockSpec((NC, BM, D), lambda i: (0, i, 0)),
          pl.BlockSpec((BM, D), lambda i: (i, 0)),
          pl.BlockSpec((NC, BM, DEGW), lambda i: (0, i, 0)),
          pl.BlockSpec((1, D), lambda i: (0, 0)),
          pl.BlockSpec((D, D), lambda i: (0, 0)),
      ],
      out_specs=[
          pl.BlockSpec((BM, D), lambda i: (i, 0)),
          pl.BlockSpec((BM, D), lambda i: (i, 0)),
      ],
      out_shape=[
          jax.ShapeDtypeStruct((N, D), jnp.float32),
          jax.ShapeDtypeStruct((N, D), jnp.float32),
      ],
  )(part, xw, degp, b, w_next)


def _tc_head(part, xw, degp, b, wstack, bstack, wco, wlin, blin):
  def body(part_ref, xw_ref, degp_ref, b_ref, ws_ref, bs_ref, wco_ref,
           wlin_ref, blin_ref, o_ref):
    dinv = _dinv_block(degp_ref)
    agg = (part_ref[0] + part_ref[1]) * dinv
    x2 = jax.nn.relu(agg + xw_ref[...] * (2.0 * dinv * dinv) + b_ref[...])

    def mm(v, k):
      return jnp.dot(v, ws_ref[k], preferred_element_type=jnp.float32)

    i1 = jax.nn.sigmoid(mm(x2, 0) + bs_ref[0])
    t1 = jnp.tanh(mm(x2, 1) + bs_ref[1])
    c1 = i1 * t1
    o1 = jax.nn.sigmoid(mm(x2, 2) + bs_ref[2] + wco_ref[0] * c1)
    h1 = jax.nn.relu(o1 * jnp.tanh(c1))
    i2 = jax.nn.sigmoid(mm(h1, 3) + bs_ref[3])
    t2 = jnp.tanh(mm(h1, 4) + bs_ref[4])
    c2 = i2 * t2
    o2 = jax.nn.sigmoid(mm(h1, 5) + bs_ref[5] + wco_ref[1] * c2)
    h2 = jax.nn.relu(o2 * jnp.tanh(c2))
    i3 = jax.nn.sigmoid(mm(h2, 6) + bs_ref[6])
    t3 = jnp.tanh(mm(h2, 7) + bs_ref[7])
    c3 = jax.nn.relu(i3 * t3)
    o_ref[...] = (jnp.dot(c3, wlin_ref[...],
                          preferred_element_type=jnp.float32)
                  + blin_ref[...])

  return pl.pallas_call(
      body,
      grid=(N // BM,),
      in_specs=[
          pl.BlockSpec((NC, BM, D), lambda i: (0, i, 0)),
          pl.BlockSpec((BM, D), lambda i: (i, 0)),
          pl.BlockSpec((NC, BM, DEGW), lambda i: (0, i, 0)),
          pl.BlockSpec((1, D), lambda i: (0, 0)),
          pl.BlockSpec((8, D, D), lambda i: (0, 0, 0)),
          pl.BlockSpec((8, 1, D), lambda i: (0, 0, 0)),
          pl.BlockSpec((2, 1, D), lambda i: (0, 0, 0)),
          pl.BlockSpec((D, 3), lambda i: (0, 0)),
          pl.BlockSpec((1, 3), lambda i: (0, 0)),
      ],
      out_specs=pl.BlockSpec((BM, 3), lambda i: (i, 0)),
      out_shape=jax.ShapeDtypeStruct((N, 3), jnp.float32),
  )(part, xw, degp, b, wstack, bstack, wco, wlin, blin)


def kernel(x, edge_index, params):
  src = edge_index[0]
  dst = edge_index[1]
  dst3 = dst.reshape(NGRP, GRP, CHUNK)
  dst3s = dst.reshape(SNGRP, SGRP, SCH)

  ones_deg = jnp.ones((CHUNK, DEGW), jnp.float32)
  zeros_deg = jnp.zeros((RS, DEGW), jnp.float32)
  zeros_rows = jnp.zeros((RS, D), jnp.float32)

  xw1 = _tc_matmul(x, params['W1'])
  degp = _sc_degree(dst3, ones_deg, zeros_deg)
  y1 = _tc_scale(xw1, degp)
  p1 = _sc_scatter(y1, src, dst3s, zeros_rows)

  b1 = params['b1'].reshape(1, D)
  b2 = params['b2'].reshape(1, D)
  xw2, y2 = _tc_combine_mm(p1, xw1, degp, b1, params['W2'])
  p2 = _sc_scatter(y2, src, dst3s, zeros_rows)

  l1, l2, l3 = params['lstm1'], params['lstm2'], params['lstm3']
  wstack = jnp.stack([
      l1['Wxi'], l1['Wxc'], l1['Wxo'],
      l2['Wxi'], l2['Wxc'], l2['Wxo'],
      l3['Wxi'], l3['Wxc'],
  ])
  bstack = jnp.stack([
      l1['bxi'] + l1['bhi'] + l1['bi'],
      l1['bxc'] + l1['bhc'] + l1['bc'],
      l1['bxo'] + l1['bho'] + l1['bo'],
      l2['bxi'] + l2['bhi'] + l2['bi'],
      l2['bxc'] + l2['bhc'] + l2['bc'],
      l2['bxo'] + l2['bho'] + l2['bo'],
      l3['bxi'] + l3['bhi'] + l3['bi'],
      l3['bxc'] + l3['bhc'] + l3['bc'],
  ]).reshape(8, 1, D)
  wco = jnp.stack([l1['wco'], l2['wco']]).reshape(2, 1, D)
  blin = params['blin'].reshape(1, 3)

  return _tc_head(p2, xw2, degp, b2, wstack, bstack, wco,
                  params['Wlin'], blin)
